# baseline (speedup 1.0000x reference)
import functools

import jax
import jax.numpy as jnp
from jax import lax
from jax.experimental import pallas as pl
from jax.experimental.pallas import tpu as pltpu
from jax.experimental.pallas import tpu_sc as plsc

HID = 2048
EPS = 1e-05

NUM_CORES = 2
NUM_SUBCORES = 16
NUM_WORKERS = NUM_CORES * NUM_SUBCORES
LANES = 16
VECS_PER_ROW = HID // LANES

CHUNK = 8
NBUF = 2
WORDVECS = HID // 32


def _dyn_gather(x, idx):
    dnums = lax.GatherDimensionNumbers(
        offset_dims=(), collapsed_slice_dims=(0,), start_index_map=(0,))
    return lax.gather(x, idx[:, None], dnums, (1,),
                      mode=lax.GatherScatterMode.PROMISE_IN_BOUNDS)


def _round_bf16_bits(x):
    u = lax.bitcast_convert_type(x, jnp.int32)
    return lax.shift_right_logical(
        u + 0x7FFF + (lax.shift_right_logical(u, 16) & 1), 16)


def _make_gather_sum(num_tokens):
    tok_per_w = num_tokens // NUM_WORKERS
    n_chunks = tok_per_w // CHUNK
    n_outer = n_chunks // NBUF
    mesh = plsc.VectorSubcoreMesh(
        core_axis_name="c", subcore_axis_name="s")

    @functools.partial(
        pl.kernel,
        out_type=jax.ShapeDtypeStruct((num_tokens, HID // 2), jnp.int32),
        mesh=mesh,
        scratch_types=[
            pltpu.VMEM((tok_per_w,), jnp.int32),
            pltpu.VMEM((tok_per_w,), jnp.int32),
            pltpu.VMEM((NBUF, CHUNK, HID), jnp.float32),
            pltpu.VMEM((NBUF, CHUNK, HID), jnp.float32),
            pltpu.VMEM((NBUF, CHUNK, HID // 2), jnp.int32),
            [pltpu.SemaphoreType.DMA] * NBUF,
            [pltpu.SemaphoreType.DMA] * NBUF,
            [pltpu.SemaphoreType.DMA] * NBUF,
        ],
    )
    def gather_sum(ids_hbm, pids_hbm, wtab_hbm, ptab_hbm, out_hbm,
                   idx_v, pidx_v, wbuf, pbuf, obuf, sem_w, sem_p, sem_o):
        wid = lax.axis_index("s") * NUM_CORES + lax.axis_index("c")
        base = wid * tok_per_w
        pltpu.sync_copy(ids_hbm.at[pl.ds(base, tok_per_w)], idx_v)
        pltpu.sync_copy(pids_hbm.at[pl.ds(base, tok_per_w)], pidx_v)

        def fire_gathers(c, b):
            off = c * CHUNK
            pltpu.async_copy(
                wtab_hbm.at[idx_v.at[pl.ds(off, CHUNK)]], wbuf.at[b],
                sem_w[b])
            pltpu.async_copy(
                ptab_hbm.at[pidx_v.at[pl.ds(off, CHUNK)]], pbuf.at[b],
                sem_p[b])

        for b in range(NBUF):
            fire_gathers(b, b)

        def outer_body(o, carry):
            for b in range(NBUF):
                c = o * NBUF + b
                pltpu.make_async_copy(
                    wtab_hbm.at[idx_v.at[pl.ds(0, CHUNK)]], wbuf.at[b],
                    sem_w[b]).wait()
                pltpu.make_async_copy(
                    ptab_hbm.at[pidx_v.at[pl.ds(0, CHUNK)]], pbuf.at[b],
                    sem_p[b]).wait()
                @pl.when(o > 0)
                def _():
                    pltpu.make_async_copy(
                        obuf.at[b], out_hbm.at[pl.ds(0, CHUNK)],
                        sem_o[b]).wait()

                iota = lax.iota(jnp.int32, LANES)
                idx_ev = (iota * 2) & 15
                idx_od = (iota * 2 + 1) & 15
                lo_half = iota < 8

                def v_body(wv):
                    sl0 = pl.ds(wv * 32, LANES)
                    sl1 = pl.ds(wv * 32 + LANES, LANES)
                    osl = pl.ds(wv * LANES, LANES)
                    for r in range(CHUNK):
                        e0 = wbuf[b, r, sl0] + pbuf[b, r, sl0]
                        e1 = wbuf[b, r, sl1] + pbuf[b, r, sl1]
                        ev = jnp.where(lo_half, _dyn_gather(e0, idx_ev),
                                       _dyn_gather(e1, idx_ev))
                        od = jnp.where(lo_half, _dyn_gather(e0, idx_od),
                                       _dyn_gather(e1, idx_od))
                        obuf[b, r, osl] = _round_bf16_bits(ev) | (
                            _round_bf16_bits(od) << 16)

                plsc.parallel_loop(0, WORDVECS, 1, unroll=2)(v_body)

                pltpu.async_copy(
                    obuf.at[b], out_hbm.at[pl.ds(base + c * CHUNK, CHUNK)],
                    sem_o[b])

                @pl.when(c + NBUF < n_chunks)
                def _():
                    fire_gathers(c + NBUF, b)
            return carry

        lax.fori_loop(0, n_outer, outer_body, 0, unroll=False)
        for b in range(NBUF):
            pltpu.make_async_copy(
                obuf.at[b], out_hbm.at[pl.ds(0, CHUNK)], sem_o[b]).wait()

    return gather_sum


def _ln_body(x_ref, t_ref, g_ref, b_ref, o_ref):
    e = x_ref[...].astype(jnp.float32) + t_ref[...]
    mu = jnp.mean(e, axis=-1, keepdims=True)
    d = e - mu
    var = jnp.mean(d * d, axis=-1, keepdims=True)
    o_ref[...] = d * lax.rsqrt(var + EPS) * g_ref[...] + b_ref[...]


def _layernorm(summed, type_row, gamma, beta, blk):
    n = summed.shape[0]
    return pl.pallas_call(
        _ln_body,
        grid=(n // blk,),
        in_specs=[
            pl.BlockSpec((blk, HID), lambda i: (i, 0)),
            pl.BlockSpec((1, HID), lambda i: (0, 0)),
            pl.BlockSpec((1, HID), lambda i: (0, 0)),
            pl.BlockSpec((1, HID), lambda i: (0, 0)),
        ],
        out_specs=pl.BlockSpec((blk, HID), lambda i: (i, 0)),
        out_shape=jax.ShapeDtypeStruct((n, HID), jnp.float32),
    )(summed, type_row, gamma, beta)


def kernel(input_ids, position_ids, word_table, pos_table, type_table,
           gamma, beta):
    b, s = input_ids.shape
    n = b * s
    packed = _make_gather_sum(n)(
        input_ids.reshape(n), position_ids.reshape(n), word_table, pos_table)
    summed = lax.bitcast_convert_type(packed, jnp.bfloat16).reshape(n, HID)
    out = _layernorm(
        summed,
        type_table[0:1, :],
        gamma.reshape(1, HID),
        beta.reshape(1, HID),
        blk=512,
    )
    return out.reshape(b, s, HID)


# speedup vs baseline: 2.6492x; 2.6492x over previous
import functools

import jax
import jax.numpy as jnp
from jax import lax
from jax.experimental import pallas as pl
from jax.experimental.pallas import tpu as pltpu
from jax.experimental.pallas import tpu_sc as plsc

HID = 2048
EPS = 1e-05

NUM_CORES = 2
NUM_SUBCORES = 16
NUM_WORKERS = NUM_CORES * NUM_SUBCORES
LANES = 16
VECS_PER_ROW = HID // LANES

CHUNK = 8
NBUF = 2
NSPLIT = 2
LN_BLK = 512


def _make_gather_sum(num_tokens):
    tok_per_w = num_tokens // NUM_WORKERS
    n_chunks = tok_per_w // CHUNK
    n_outer = n_chunks // NBUF
    mesh = plsc.VectorSubcoreMesh(
        core_axis_name="c", subcore_axis_name="s")

    @functools.partial(
        pl.kernel,
        out_type=jax.ShapeDtypeStruct((num_tokens, HID), jnp.float32),
        mesh=mesh,
        scratch_types=[
            pltpu.VMEM((tok_per_w,), jnp.int32),
            pltpu.VMEM((tok_per_w,), jnp.int32),
            pltpu.VMEM((NBUF, CHUNK, HID), jnp.float32),
            pltpu.VMEM((NBUF, CHUNK, HID), jnp.float32),
            pltpu.VMEM((NBUF, CHUNK, HID), jnp.float32),
            [pltpu.SemaphoreType.DMA] * NBUF,
            [pltpu.SemaphoreType.DMA] * NBUF,
            [pltpu.SemaphoreType.DMA] * NBUF,
        ],
    )
    def gather_sum(ids_hbm, pids_hbm, wtab_hbm, ptab_hbm, out_hbm,
                   idx_v, pidx_v, wbuf, pbuf, obuf, sem_w, sem_p, sem_o):
        wid = lax.axis_index("s") * NUM_CORES + lax.axis_index("c")
        base = wid * tok_per_w
        pltpu.sync_copy(ids_hbm.at[pl.ds(base, tok_per_w)], idx_v)
        pltpu.sync_copy(pids_hbm.at[pl.ds(base, tok_per_w)], pidx_v)

        def fire_gathers(c, b):
            off = c * CHUNK
            pltpu.async_copy(
                wtab_hbm.at[idx_v.at[pl.ds(off, CHUNK)]], wbuf.at[b],
                sem_w[b])
            pltpu.async_copy(
                ptab_hbm.at[pidx_v.at[pl.ds(off, CHUNK)]], pbuf.at[b],
                sem_p[b])

        for b in range(NBUF):
            fire_gathers(b, b)

        def outer_body(o, carry):
            for b in range(NBUF):
                c = o * NBUF + b
                pltpu.make_async_copy(
                    wtab_hbm.at[idx_v.at[pl.ds(0, CHUNK)]], wbuf.at[b],
                    sem_w[b]).wait()
                pltpu.make_async_copy(
                    ptab_hbm.at[pidx_v.at[pl.ds(0, CHUNK)]], pbuf.at[b],
                    sem_p[b]).wait()
                @pl.when(o > 0)
                def _():
                    pltpu.make_async_copy(
                        obuf.at[b], out_hbm.at[pl.ds(0, CHUNK)],
                        sem_o[b]).wait()

                def v_body(v):
                    sl = pl.ds(v * LANES, LANES)
                    for r in range(CHUNK):
                        obuf[b, r, sl] = wbuf[b, r, sl] + pbuf[b, r, sl]

                plsc.parallel_loop(0, VECS_PER_ROW, 1, unroll=4)(v_body)

                pltpu.async_copy(
                    obuf.at[b], out_hbm.at[pl.ds(base + c * CHUNK, CHUNK)],
                    sem_o[b])

                @pl.when(c + NBUF < n_chunks)
                def _():
                    fire_gathers(c + NBUF, b)
            return carry

        lax.fori_loop(0, n_outer, outer_body, 0, unroll=False)
        for b in range(NBUF):
            pltpu.make_async_copy(
                obuf.at[b], out_hbm.at[pl.ds(0, CHUNK)], sem_o[b]).wait()

    return gather_sum


def _ln_body(x_ref, t_ref, g_ref, b_ref, o_ref):
    e = x_ref[...] + t_ref[...]
    mu = jnp.mean(e, axis=-1, keepdims=True)
    d = e - mu
    var = jnp.mean(d * d, axis=-1, keepdims=True)
    o_ref[...] = d * lax.rsqrt(var + EPS) * g_ref[...] + b_ref[...]


def _ln_body_acc(x_ref, t_ref, g_ref, b_ref, buf_ref, o_ref):
    del buf_ref
    _ln_body(x_ref, t_ref, g_ref, b_ref, o_ref)


def _layernorm_slice(summed, type_row, gamma, beta, n_total, blk_off, buf):
    n = summed.shape[0]
    in_specs = [
        pl.BlockSpec((LN_BLK, HID), lambda i: (i, 0)),
        pl.BlockSpec((1, HID), lambda i: (0, 0)),
        pl.BlockSpec((1, HID), lambda i: (0, 0)),
        pl.BlockSpec((1, HID), lambda i: (0, 0)),
    ]
    args = [summed, type_row, gamma, beta]
    kwargs = {}
    body = _ln_body
    if buf is not None:
        in_specs.append(pl.BlockSpec(memory_space=pl.ANY))
        args.append(buf)
        kwargs["input_output_aliases"] = {4: 0}
        body = _ln_body_acc
    return pl.pallas_call(
        body,
        grid=(n // LN_BLK,),
        in_specs=in_specs,
        out_specs=pl.BlockSpec((LN_BLK, HID), lambda i: (i + blk_off, 0)),
        out_shape=jax.ShapeDtypeStruct((n_total, HID), jnp.float32),
        **kwargs,
    )(*args)


def kernel(input_ids, position_ids, word_table, pos_table, type_table,
           gamma, beta):
    b, s = input_ids.shape
    n = b * s
    ids = input_ids.reshape(n)
    pids = position_ids.reshape(n)
    h = n // NSPLIT
    gs = _make_gather_sum(h)
    type_row = type_table[0:1, :]
    g2 = gamma.reshape(1, HID)
    b2 = beta.reshape(1, HID)

    summed = [
        gs(ids[q * h:(q + 1) * h], pids[q * h:(q + 1) * h],
           word_table, pos_table)
        for q in range(NSPLIT)
    ]
    buf = None
    for q in range(NSPLIT):
        buf = _layernorm_slice(
            summed[q], type_row, g2, b2, n, q * (h // LN_BLK), buf)
    return buf.reshape(b, s, HID)
